# baseline (speedup 1.0000x reference)
import functools

import jax
import jax.numpy as jnp
from jax import lax
from jax.experimental import pallas as pl
from jax.experimental.pallas import tpu as pltpu
from jax.experimental.pallas import tpu_sc as plsc

_L = 16


def _emb_kernel(B, T, V, D, NC, NS):
    NW = NC * NS
    TCH = T // NW
    mesh = plsc.VectorSubcoreMesh(core_axis_name="c", subcore_axis_name="s")

    @functools.partial(
        pl.kernel,
        mesh=mesh,
        out_type=jax.ShapeDtypeStruct((B * T, D), jnp.float32),
        scratch_types=[
            pltpu.VMEM((TCH,), jnp.int32),
            pltpu.VMEM((TCH, D), jnp.float32),
            pltpu.VMEM((TCH, D), jnp.float32),
            pltpu.SemaphoreType.DMA,
        ],
    )
    def body(idx_hbm, tok_hbm, pos_hbm, out_hbm, idx_v, rows_v, pos_v, sem):
        wid = lax.axis_index("s") * NC + lax.axis_index("c")
        t0 = wid * TCH
        pltpu.sync_copy(pos_hbm.at[pl.ds(t0, TCH)], pos_v)

        def per_batch(b, carry):
            pltpu.sync_copy(idx_hbm.at[b, pl.ds(t0, TCH)], idx_v)
            pltpu.async_copy(tok_hbm.at[idx_v], rows_v, sem).wait()

            def per_row(r, carry2):
                for j in range(D // _L):
                    sl = pl.ds(j * _L, _L)
                    rows_v[r, sl] = rows_v[r, sl] + pos_v[r, sl]
                return carry2

            lax.fori_loop(0, TCH, per_row, 0)
            pltpu.sync_copy(rows_v, out_hbm.at[pl.ds(b * T + t0, TCH)])
            return carry

        lax.fori_loop(0, B, per_batch, 0)

    return body


def kernel(idx, tok_table, pos_table):
    B, T = idx.shape
    V, D = tok_table.shape
    info = plsc.get_sparse_core_info()
    NC, NS = info.num_cores, info.num_subcores
    fn = _emb_kernel(B, T, V, D, NC, NS)
    out = fn(idx.astype(jnp.int32), tok_table, pos_table)
    return out.reshape(B, T, D)


# speedup vs baseline: 1.1941x; 1.1941x over previous
import functools

import jax
import jax.numpy as jnp
from jax import lax
from jax.experimental import pallas as pl
from jax.experimental.pallas import tpu as pltpu
from jax.experimental.pallas import tpu_sc as plsc

_L = 16


def _emb_kernel(B, T, V, D, NC, NS):
    NW = NC * NS
    TCH = T // NW
    mesh = plsc.VectorSubcoreMesh(core_axis_name="c", subcore_axis_name="s")

    @functools.partial(
        pl.kernel,
        mesh=mesh,
        out_type=jax.ShapeDtypeStruct((B * T, D), jnp.float32),
        scratch_types=[
            pltpu.VMEM((B, TCH), jnp.int32),
            pltpu.VMEM((TCH, D), jnp.float32),
            pltpu.VMEM((TCH, D), jnp.float32),
            pltpu.VMEM((TCH, D), jnp.float32),
            pltpu.SemaphoreType.DMA,
            pltpu.SemaphoreType.DMA,
            pltpu.SemaphoreType.DMA,
            pltpu.SemaphoreType.DMA,
        ],
    )
    def body(idx_hbm, tok_hbm, pos_hbm, out_hbm, idx_v, pos_v, rows0, rows1,
             g0, g1, w0, w1):
        wid = lax.axis_index("s") * NC + lax.axis_index("c")
        t0 = wid * TCH
        for b in range(B):
            pltpu.sync_copy(idx_hbm.at[b, pl.ds(t0, TCH)], idx_v.at[b])
        pltpu.sync_copy(pos_hbm.at[pl.ds(t0, TCH)], pos_v)

        rows = (rows0, rows1)
        gsem = (g0, g1)
        wsem = (w0, w1)
        gd = [None, None]
        wd = [None, None]
        gd[0] = pltpu.async_copy(tok_hbm.at[idx_v.at[0]], rows0, g0)
        for b in range(B):
            cur = b % 2
            nxt = 1 - cur
            if b + 1 < B:
                if wd[nxt] is not None:
                    wd[nxt].wait()
                gd[nxt] = pltpu.async_copy(
                    tok_hbm.at[idx_v.at[b + 1]], rows[nxt], gsem[nxt])
            gd[cur].wait()
            rcur = rows[cur]

            def per_row(r, carry, rcur=rcur):
                for j in range(D // _L):
                    sl = pl.ds(j * _L, _L)
                    plsc.addupdate(rcur.at[r, sl], pos_v[r, sl])
                return carry

            lax.fori_loop(0, TCH, per_row, 0)
            wd[cur] = pltpu.async_copy(
                rcur, out_hbm.at[pl.ds(b * T + t0, TCH)], wsem[cur])
        wd[0].wait()
        wd[1].wait()

    return body


def kernel(idx, tok_table, pos_table):
    B, T = idx.shape
    V, D = tok_table.shape
    info = plsc.get_sparse_core_info()
    NC, NS = info.num_cores, info.num_subcores
    fn = _emb_kernel(B, T, V, D, NC, NS)
    out = fn(idx.astype(jnp.int32), tok_table, pos_table)
    return out.reshape(B, T, D)


# speedup vs baseline: 1.3180x; 1.1038x over previous
import functools

import jax
import jax.numpy as jnp
from jax import lax
from jax.experimental import pallas as pl
from jax.experimental.pallas import tpu as pltpu
from jax.experimental.pallas import tpu_sc as plsc

_L = 16
_NBUF = 3


def _emb_kernel(B, T, V, D, NC, NS):
    NW = NC * NS
    TCH = T // NW
    mesh = plsc.VectorSubcoreMesh(core_axis_name="c", subcore_axis_name="s")

    @functools.partial(
        pl.kernel,
        mesh=mesh,
        out_type=jax.ShapeDtypeStruct((B * T, D), jnp.float32),
        scratch_types=(
            [pltpu.VMEM((B, TCH), jnp.int32),
             pltpu.VMEM((TCH, D), jnp.float32)]
            + [pltpu.VMEM((TCH, D), jnp.float32) for _ in range(_NBUF)]
            + [pltpu.SemaphoreType.DMA for _ in range(2 * _NBUF + 1)]
        ),
    )
    def body(idx_hbm, tok_hbm, pos_hbm, out_hbm, idx_v, pos_v, *bufs_sems):
        rows = bufs_sems[:_NBUF]
        gsem = bufs_sems[_NBUF:2 * _NBUF]
        wsem = bufs_sems[2 * _NBUF:3 * _NBUF]
        psem = bufs_sems[3 * _NBUF]
        wid = lax.axis_index("s") * NC + lax.axis_index("c")
        t0 = wid * TCH

        idx_d = [pltpu.async_copy(idx_hbm.at[b, pl.ds(t0, TCH)],
                                  idx_v.at[b], psem) for b in range(B)]
        pos_d = pltpu.async_copy(pos_hbm.at[pl.ds(t0, TCH)], pos_v, psem)
        for d in idx_d:
            d.wait()

        def gather(b):
            return pltpu.async_copy(
                tok_hbm.at[idx_v.at[b]], rows[b % _NBUF], gsem[b % _NBUF])

        gd = [None] * _NBUF
        wd = [None] * _NBUF
        for b in range(min(2, B)):
            gd[b % _NBUF] = gather(b)
        pos_d.wait()

        for b in range(B):
            cur = b % _NBUF
            if b + 2 < B:
                nb = (b + 2) % _NBUF
                if wd[nb] is not None:
                    wd[nb].wait()
                gd[nb] = gather(b + 2)
            gd[cur].wait()
            rcur = rows[cur]

            def per_row(r, carry, rcur=rcur):
                for j in range(D // _L):
                    sl = pl.ds(j * _L, _L)
                    plsc.addupdate(rcur.at[r, sl], pos_v[r, sl])
                return carry

            lax.fori_loop(0, TCH, per_row, 0)
            wd[cur] = pltpu.async_copy(
                rcur, out_hbm.at[pl.ds(b * T + t0, TCH)], wsem[cur])
        for d in wd:
            if d is not None:
                d.wait()

    return body


def kernel(idx, tok_table, pos_table):
    B, T = idx.shape
    V, D = tok_table.shape
    info = plsc.get_sparse_core_info()
    NC, NS = info.num_cores, info.num_subcores
    fn = _emb_kernel(B, T, V, D, NC, NS)
    out = fn(idx.astype(jnp.int32), tok_table, pos_table)
    return out.reshape(B, T, D)
